# baseline (speedup 1.0000x reference)
import jax
import jax.numpy as jnp
from jax import lax
from jax.experimental import pallas as pl
from jax.experimental.pallas import tpu as pltpu
from jax.experimental.pallas import tpu_sc as plsc

VOCAB = 1024
N_ROWS = 32768
NUM_CORES = 2
NUM_SUBCORES = 16
NW = NUM_CORES * NUM_SUBCORES
ROWS_PER_TILE = N_ROWS // NW
LANES = 16
IDX_CHUNK = 128
K = ROWS_PER_TILE // IDX_CHUNK
TILE_ELEMS = ROWS_PER_TILE * VOCAB


def _sc_onehot(ids_hbm, fill_hbm, zeros_hbm, out_hbm,
               idx_v, pos_v, src_v, zsem, ssem):
    c = lax.axis_index("c")
    s = lax.axis_index("s")
    wid = s * NUM_CORES + c
    base = wid * ROWS_PER_TILE
    zero_dma = pltpu.async_copy(
        zeros_hbm, out_hbm.at[pl.ds(base * VOCAB, TILE_ELEMS)], zsem)
    pltpu.sync_copy(ids_hbm.at[pl.ds(base, ROWS_PER_TILE)], idx_v)
    pltpu.sync_copy(fill_hbm, src_v)
    lane = lax.iota(jnp.int32, LANES)
    for j in range(K):
        for t in range(IDX_CHUNK // LANES):
            off = j * IDX_CHUNK + t * LANES
            col = idx_v[pl.ds(off, LANES)] % VOCAB
            row = base + off + lane
            pos_v[j, pl.ds(t * LANES, LANES)] = row * VOCAB + col
    zero_dma.wait()
    handles = []
    for j in range(K):
        handles.append(pltpu.async_copy(src_v, out_hbm.at[pos_v.at[j]], ssem))
    for h in handles:
        h.wait()


def kernel(input_ids, fill_value):
    bs, seq = input_ids.shape
    ids = input_ids.reshape(N_ROWS)
    fill128 = jnp.broadcast_to(fill_value.astype(jnp.float32), (IDX_CHUNK,))
    zeros = jnp.zeros((TILE_ELEMS,), jnp.float32)
    mesh = plsc.VectorSubcoreMesh(core_axis_name="c", subcore_axis_name="s")
    f = pl.kernel(
        _sc_onehot,
        out_type=jax.ShapeDtypeStruct((N_ROWS * VOCAB,), jnp.float32),
        mesh=mesh,
        compiler_params=pltpu.CompilerParams(needs_layout_passes=False),
        scratch_types=[
            pltpu.VMEM((ROWS_PER_TILE,), jnp.int32),
            pltpu.VMEM((K, IDX_CHUNK), jnp.int32),
            pltpu.VMEM((IDX_CHUNK,), jnp.float32),
            pltpu.SemaphoreType.DMA,
            pltpu.SemaphoreType.DMA,
        ],
    )
    out = f(ids, fill128, zeros)
    return out.reshape(bs, seq, VOCAB)


# speedup vs baseline: 20.0469x; 20.0469x over previous
import jax
import jax.numpy as jnp
from jax import lax
from jax.experimental import pallas as pl
from jax.experimental.pallas import tpu as pltpu
from jax.experimental.pallas import tpu_sc as plsc

VOCAB = 1024
N_ROWS = 32768
NUM_CORES = 2
NUM_SUBCORES = 16
NW = NUM_CORES * NUM_SUBCORES
ROWS_PER_TILE = N_ROWS // NW
CHUNK = 32
NCHUNK = ROWS_PER_TILE // CHUNK
LANES = 16
NBUF = 2


def _sc_onehot(ids_hbm, fill_hbm, zeros_hbm, out_hbm,
               idx_v, fill_v, buf0, buf1, sem0, sem1):
    c = lax.axis_index("c")
    s = lax.axis_index("s")
    wid = s * NUM_CORES + c
    base = wid * ROWS_PER_TILE
    pltpu.sync_copy(ids_hbm.at[pl.ds(base, ROWS_PER_TILE)], idx_v)
    pltpu.sync_copy(fill_hbm, fill_v)
    pltpu.sync_copy(zeros_hbm, buf0)
    pltpu.sync_copy(zeros_hbm, buf1)
    fill = fill_v[...]
    zero = jnp.zeros((LANES,), jnp.float32)
    lane = lax.iota(jnp.int32, LANES)
    bufs = (buf0, buf1)
    sems = (sem0, sem1)
    handles = [None, None]

    def scatter_chunk(buf, j, val):
        for t in range(CHUNK // LANES):
            cols = idx_v[pl.ds(j * CHUNK + t * LANES, LANES)] % VOCAB
            flat = (lane + t * LANES) * VOCAB + cols
            plsc.store_scatter(buf, [flat], val)

    for j in range(NCHUNK):
        b = j % NBUF
        buf = bufs[b]
        if handles[b] is not None:
            handles[b].wait()
            scatter_chunk(buf, j - NBUF, zero)
        scatter_chunk(buf, j, fill)
        handles[b] = pltpu.async_copy(
            buf, out_hbm.at[pl.ds((base + j * CHUNK) * VOCAB, CHUNK * VOCAB)],
            sems[b])
    for h in handles:
        h.wait()


def kernel(input_ids, fill_value):
    bs, seq = input_ids.shape
    ids = input_ids.reshape(N_ROWS)
    fillv = jnp.broadcast_to(fill_value.astype(jnp.float32), (LANES,))
    zeros = jnp.zeros((CHUNK * VOCAB,), jnp.float32)
    mesh = plsc.VectorSubcoreMesh(core_axis_name="c", subcore_axis_name="s")
    f = pl.kernel(
        _sc_onehot,
        out_type=jax.ShapeDtypeStruct((N_ROWS * VOCAB,), jnp.float32),
        mesh=mesh,
        compiler_params=pltpu.CompilerParams(needs_layout_passes=False),
        scratch_types=[
            pltpu.VMEM((ROWS_PER_TILE,), jnp.int32),
            pltpu.VMEM((LANES,), jnp.float32),
            pltpu.VMEM((CHUNK * VOCAB,), jnp.float32),
            pltpu.VMEM((CHUNK * VOCAB,), jnp.float32),
            pltpu.SemaphoreType.DMA,
            pltpu.SemaphoreType.DMA,
        ],
    )
    out = f(ids, fillv, zeros)
    return out.reshape(bs, seq, VOCAB)
